# Initial kernel scaffold; baseline (speedup 1.0000x reference)
import jax
import jax.numpy as jnp
from jax.experimental import pallas as pl


def kernel(token_ids, weight):
    raise NotImplementedError("write your pallas kernel here")


# speedup vs baseline: 1.8346x; 1.8346x over previous
import functools

import jax
import jax.numpy as jnp
from jax import lax
from jax.experimental import pallas as pl
from jax.experimental.pallas import tpu as pltpu
from jax.experimental.pallas import tpu_sc as plsc

_LANES = 128
_GATHERS_PER_CHUNK = 8
_CHUNK = _LANES * _GATHERS_PER_CHUNK


@functools.cache
def _build(num_idx_rows: int, dim: int):
    mesh = plsc.VectorSubcoreMesh(core_axis_name="c", subcore_axis_name="s")
    num_workers = mesh.num_cores * mesh.num_subcores
    rows_per_worker = num_idx_rows // num_workers
    chunks = rows_per_worker // _GATHERS_PER_CHUNK
    nc = mesh.num_cores

    @functools.partial(
        pl.kernel,
        out_type=jax.ShapeDtypeStruct((num_idx_rows * _LANES, dim), jnp.float32),
        mesh=mesh,
        scratch_types=[
            pltpu.VMEM((_GATHERS_PER_CHUNK, _LANES), jnp.int32),
            pltpu.VMEM((_CHUNK, dim), jnp.float32),
            pltpu.SemaphoreType.DMA,
        ],
        compiler_params=pltpu.CompilerParams(use_tc_tiling_on_sc=False),
    )
    def body(idx_hbm, table_hbm, out_hbm, idx_v, rows_v, sem):
        wid = lax.axis_index("s") * nc + lax.axis_index("c")
        row_base = wid * rows_per_worker

        @pl.loop(0, chunks)
        def _chunk(ci):
            r0 = row_base + ci * _GATHERS_PER_CHUNK
            pltpu.sync_copy(idx_hbm.at[pl.ds(r0, _GATHERS_PER_CHUNK)], idx_v)
            descs = [
                pltpu.async_copy(
                    table_hbm.at[idx_v.at[j]],
                    rows_v.at[pl.ds(j * _LANES, _LANES)],
                    sem,
                )
                for j in range(_GATHERS_PER_CHUNK)
            ]
            for d in descs:
                d.wait()
            pltpu.sync_copy(rows_v, out_hbm.at[pl.ds(r0 * _LANES, _CHUNK)])

    return body


def kernel(token_ids, weight):
    orig_shape = token_ids.shape
    dim = weight.shape[1]
    flat = token_ids.reshape(-1).astype(jnp.int32)
    idx2d = flat.reshape(flat.shape[0] // _LANES, _LANES)
    out = _build(idx2d.shape[0], dim)(idx2d, weight)
    return out.reshape(*orig_shape, dim)


# speedup vs baseline: 1.8632x; 1.0156x over previous
import functools

import jax
import jax.numpy as jnp
from jax import lax
from jax.experimental import pallas as pl
from jax.experimental.pallas import tpu as pltpu
from jax.experimental.pallas import tpu_sc as plsc

_LANES = 128
_G = 4
_CHUNK = _LANES * _G
_NBUF = 2


@functools.cache
def _build(num_idx_rows: int, dim: int):
    mesh = plsc.VectorSubcoreMesh(core_axis_name="c", subcore_axis_name="s")
    num_workers = mesh.num_cores * mesh.num_subcores
    rows_per_worker = num_idx_rows // num_workers
    chunks = rows_per_worker // _G
    nc = mesh.num_cores

    @functools.partial(
        pl.kernel,
        out_type=jax.ShapeDtypeStruct((num_idx_rows * _LANES, dim), jnp.float32),
        mesh=mesh,
        scratch_types=[
            pltpu.VMEM((_G, _LANES), jnp.int32),
            pltpu.VMEM((_G, _LANES), jnp.int32),
            pltpu.VMEM((_CHUNK, dim), jnp.float32),
            pltpu.VMEM((_CHUNK, dim), jnp.float32),
            pltpu.SemaphoreType.DMA,
            pltpu.SemaphoreType.DMA,
            pltpu.SemaphoreType.DMA,
            pltpu.SemaphoreType.DMA,
        ],
        compiler_params=pltpu.CompilerParams(use_tc_tiling_on_sc=False),
    )
    def body(idx_hbm, table_hbm, out_hbm, idx0, idx1, rows0, rows1,
             g0, g1, s0, s1):
        bufs = ((idx0, rows0, g0, s0), (idx1, rows1, g1, s1))
        wid = lax.axis_index("s") * nc + lax.axis_index("c")
        row_base = wid * rows_per_worker

        @pl.loop(0, chunks * _G, step=_NBUF * _G)
        def _outer(c0):
            descs = []
            for b in range(_NBUF):
                idx_v, rows_v, gsem, ssem = bufs[b]
                r0 = row_base + c0 + b * _G

                @pl.when(c0 > 0)
                def _drain():
                    pltpu.make_async_copy(
                        rows_v, out_hbm.at[pl.ds(0, _CHUNK)], ssem).wait()

                pltpu.sync_copy(idx_hbm.at[pl.ds(r0, _G)], idx_v)
                descs.append([
                    pltpu.async_copy(
                        table_hbm.at[idx_v.at[j]],
                        rows_v.at[pl.ds(j * _LANES, _LANES)],
                        gsem,
                    )
                    for j in range(_G)
                ])
            for b in range(_NBUF):
                idx_v, rows_v, gsem, ssem = bufs[b]
                for d in descs[b]:
                    d.wait()
                r0 = row_base + c0 + b * _G
                pltpu.async_copy(
                    rows_v, out_hbm.at[pl.ds(r0 * _LANES, _CHUNK)], ssem)

        for b in range(_NBUF):
            _, rows_v, _, ssem = bufs[b]
            pltpu.make_async_copy(
                rows_v, out_hbm.at[pl.ds(0, _CHUNK)], ssem).wait()

    return body


def kernel(token_ids, weight):
    orig_shape = token_ids.shape
    dim = weight.shape[1]
    flat = token_ids.reshape(-1).astype(jnp.int32)
    idx2d = flat.reshape(flat.shape[0] // _LANES, _LANES)
    out = _build(idx2d.shape[0], dim)(idx2d, weight)
    return out.reshape(*orig_shape, dim)
